# Initial kernel scaffold; baseline (speedup 1.0000x reference)
import jax
import jax.numpy as jnp
from jax.experimental import pallas as pl


def kernel(x, item_id_table):
    raise NotImplementedError("write your pallas kernel here")


# speedup vs baseline: 2.7619x; 2.7619x over previous
import functools

import jax
import jax.numpy as jnp
from jax import lax
from jax.experimental import pallas as pl
from jax.experimental.pallas import tpu as pltpu
from jax.experimental.pallas import tpu_sc as plsc

VOCAB = 1000
EMBED_DIM = 32
BATCH = 16384
HIST = 50
B = BATCH * HIST

NUM_CORES = 2
NUM_SUBCORES = 16
NW = NUM_CORES * NUM_SUBCORES
BPW = B // NW

IDX_ROW = 128
ROWS_PER_WORKER = BPW // IDX_ROW
GROUPS_PER_CHUNK = 8
CHUNK = GROUPS_PER_CHUNK * IDX_ROW
NCHUNK = BPW // CHUNK

_mesh = plsc.VectorSubcoreMesh(core_axis_name="c", subcore_axis_name="s")


@functools.partial(
    pl.kernel,
    mesh=_mesh,
    out_type=jax.ShapeDtypeStruct((B, EMBED_DIM), jnp.float32),
    compiler_params=pltpu.CompilerParams(use_tc_tiling_on_sc=False),
    scratch_types=[
        pltpu.VMEM((ROWS_PER_WORKER, IDX_ROW), jnp.int32),
        pltpu.VMEM((2, CHUNK, EMBED_DIM), jnp.float32),
        pltpu.SemaphoreType.DMA,
        pltpu.SemaphoreType.DMA,
        pltpu.SemaphoreType.DMA,
        pltpu.SemaphoreType.DMA,
    ],
)
def _sc_embed(idx_hbm, table_hbm, out_hbm, idx_v, rows_v, g0, g1, w0, w1):
    wid = lax.axis_index("s") * NUM_CORES + lax.axis_index("c")
    row_base = wid * ROWS_PER_WORKER
    out_base = wid * BPW

    pltpu.sync_copy(idx_hbm.at[pl.ds(row_base, ROWS_PER_WORKER)], idx_v)

    gsems = (g0, g1)
    wsems = (w0, w1)
    write_handles = [None, None]
    for c in range(NCHUNK):
        buf = c % 2
        if write_handles[buf] is not None:
            write_handles[buf].wait()
        gh = []
        for g in range(GROUPS_PER_CHUNK):
            gh.append(
                pltpu.async_copy(
                    table_hbm.at[idx_v.at[c * GROUPS_PER_CHUNK + g]],
                    rows_v.at[buf, pl.ds(g * IDX_ROW, IDX_ROW)],
                    gsems[buf],
                )
            )
        for h in gh:
            h.wait()
        write_handles[buf] = pltpu.async_copy(
            rows_v.at[buf],
            out_hbm.at[pl.ds(out_base + c * CHUNK, CHUNK)],
            wsems[buf],
        )
    for h in write_handles:
        if h is not None:
            h.wait()


def kernel(x, item_id_table):
    idx2d = x.reshape(B // IDX_ROW, IDX_ROW)
    out = _sc_embed(idx2d, item_id_table)
    return out.reshape(BATCH, HIST, EMBED_DIM)


# speedup vs baseline: 2.7800x; 1.0066x over previous
import functools

import jax
import jax.numpy as jnp
from jax import lax
from jax.experimental import pallas as pl
from jax.experimental.pallas import tpu as pltpu
from jax.experimental.pallas import tpu_sc as plsc

VOCAB = 1000
EMBED_DIM = 32
BATCH = 16384
HIST = 50
B = BATCH * HIST

NUM_CORES = 2
NUM_SUBCORES = 16
NW = NUM_CORES * NUM_SUBCORES
BPW = B // NW

CHUNK = 1024
NCHUNK = BPW // CHUNK
GROUP = 16

_mesh = plsc.VectorSubcoreMesh(core_axis_name="c", subcore_axis_name="s")


@functools.partial(
    pl.kernel,
    mesh=_mesh,
    out_type=jax.ShapeDtypeStruct((B, EMBED_DIM), jnp.float32),
    compiler_params=pltpu.CompilerParams(use_tc_tiling_on_sc=False),
    scratch_types=[
        pltpu.VMEM((BPW,), jnp.int32),
        pltpu.VMEM((VOCAB, EMBED_DIM), jnp.float32),
        pltpu.VMEM((2, CHUNK, EMBED_DIM), jnp.float32),
        pltpu.SemaphoreType.DMA,
        pltpu.SemaphoreType.DMA,
    ],
)
def _sc_embed(idx_hbm, table_hbm, out_hbm, idx_v, table_v, buf, w0, w1):
    wid = lax.axis_index("s") * NUM_CORES + lax.axis_index("c")
    base = wid * BPW

    pltpu.sync_copy(table_hbm, table_v)
    pltpu.sync_copy(idx_hbm.at[pl.ds(base, BPW)], idx_v)

    wsems = (w0, w1)
    write_handles = [None, None]
    half = EMBED_DIM // 2
    for c in range(NCHUNK):
        bsel = c % 2
        if write_handles[bsel] is not None:
            write_handles[bsel].wait()

        def row_body(g, carry, _c=c, _b=bsel):
            r0 = g * GROUP
            iv = idx_v[pl.ds(_c * CHUNK + r0, GROUP)]
            for u in range(GROUP):
                r = r0 + u
                i = iv[u]
                buf[_b, r, pl.ds(0, half)] = table_v[i, pl.ds(0, half)]
                buf[_b, r, pl.ds(half, half)] = table_v[i, pl.ds(half, half)]
            return carry

        lax.fori_loop(0, CHUNK // GROUP, row_body, 0)
        write_handles[bsel] = pltpu.async_copy(
            buf.at[bsel],
            out_hbm.at[pl.ds(base + c * CHUNK, CHUNK)],
            wsems[bsel],
        )
    for h in write_handles:
        if h is not None:
            h.wait()


def kernel(x, item_id_table):
    out = _sc_embed(x.reshape(B), item_id_table)
    return out.reshape(BATCH, HIST, EMBED_DIM)


# speedup vs baseline: 7.0573x; 2.5386x over previous
import functools

import jax
import jax.numpy as jnp
from jax import lax
from jax.experimental import pallas as pl
from jax.experimental.pallas import tpu as pltpu
from jax.experimental.pallas import tpu_sc as plsc

VOCAB = 1000
EMBED_DIM = 32
BATCH = 16384
HIST = 50
B = BATCH * HIST
LANE = 128
TROWS = VOCAB * EMBED_DIM // LANE
OROWS = B * EMBED_DIM // LANE
PER_ROW = LANE // EMBED_DIM

NUM_CORES = 2
NUM_SUBCORES = 16
NW = NUM_CORES * NUM_SUBCORES
BPW = B // NW

CHUNK = 1024
NCHUNK = BPW // CHUNK
NPAIR = NCHUNK // 2
GROUP = 16
ROWS_PER_CHUNK = CHUNK // PER_ROW
HALF = EMBED_DIM // 2

_mesh = plsc.VectorSubcoreMesh(core_axis_name="c", subcore_axis_name="s")


@functools.partial(
    pl.kernel,
    mesh=_mesh,
    out_type=jax.ShapeDtypeStruct((OROWS, LANE), jnp.float32),
    scratch_types=[
        pltpu.VMEM((BPW,), jnp.int32),
        pltpu.VMEM((TROWS, LANE), jnp.float32),
        pltpu.VMEM((2, ROWS_PER_CHUNK, LANE), jnp.float32),
        pltpu.SemaphoreType.DMA,
        pltpu.SemaphoreType.DMA,
    ],
)
def _sc_embed(idx_hbm, table_hbm, out_hbm, idx_v, table_v, buf, w0, w1):
    wid = lax.axis_index("s") * NUM_CORES + lax.axis_index("c")
    base = wid * BPW
    out_row_base = wid * (BPW // PER_ROW)

    pltpu.sync_copy(table_hbm, table_v)
    pltpu.sync_copy(idx_hbm.at[pl.ds(base, BPW)], idx_v)

    def gather_chunk(c, bsel):
        @plsc.parallel_loop(0, CHUNK // GROUP, unroll=2)
        def row_body(g):
            iv = idx_v[pl.ds(c * CHUNK + g * GROUP, GROUP)]
            br0 = g * (GROUP // PER_ROW)
            for u in range(GROUP):
                i = iv[u]
                tq = i >> 2
                tcol = (i & 3) * EMBED_DIM
                br = br0 + (u // PER_ROW)
                bcol = (u % PER_ROW) * EMBED_DIM
                buf[bsel, br, pl.ds(bcol, HALF)] = table_v[tq, pl.ds(tcol, HALF)]
                buf[bsel, br, pl.ds(bcol + HALF, HALF)] = table_v[
                    tq, pl.ds(tcol + HALF, HALF)]

    def write_chunk(c, bsel, sem):
        pltpu.async_copy(
            buf.at[bsel],
            out_hbm.at[pl.ds(out_row_base + c * ROWS_PER_CHUNK, ROWS_PER_CHUNK)],
            sem,
        )

    def drain(sem):
        pltpu.make_async_copy(
            buf.at[0], out_hbm.at[pl.ds(out_row_base, ROWS_PER_CHUNK)], sem
        ).wait()

    def pair_body(p, carry):
        c0 = p * 2

        @pl.when(p > 0)
        def _():
            drain(w0)

        gather_chunk(c0, 0)
        write_chunk(c0, 0, w0)

        @pl.when(p > 0)
        def _():
            drain(w1)

        gather_chunk(c0 + 1, 1)
        write_chunk(c0 + 1, 1, w1)
        return carry

    lax.fori_loop(0, NPAIR, pair_body, 0)

    drain(w0)
    gather_chunk(NPAIR * 2, 0)
    write_chunk(NPAIR * 2, 0, w0)
    drain(w1)
    drain(w0)


def kernel(x, item_id_table):
    out = _sc_embed(x.reshape(B), item_id_table.reshape(TROWS, LANE))
    return out.reshape(BATCH, HIST, EMBED_DIM)


# speedup vs baseline: 28.8986x; 4.0949x over previous
import functools

import jax
import jax.numpy as jnp
from jax import lax
from jax.experimental import pallas as pl
from jax.experimental.pallas import tpu as pltpu
from jax.experimental.pallas import tpu_sc as plsc

VOCAB = 1000
EMBED_DIM = 32
BATCH = 16384
HIST = 50
OROWS = HIST * EMBED_DIM

NUM_CORES = 2
NUM_SUBCORES = 16
NW = NUM_CORES * NUM_SUBCORES
COLS = BATCH // NW
NGROUP = COLS // 16
NPAIR = HIST // 2

_mesh = plsc.VectorSubcoreMesh(core_axis_name="c", subcore_axis_name="s")


@functools.partial(
    pl.kernel,
    mesh=_mesh,
    out_type=jax.ShapeDtypeStruct((OROWS, BATCH), jnp.float32),
    compiler_params=pltpu.CompilerParams(needs_layout_passes=False),
    scratch_types=[
        pltpu.VMEM((EMBED_DIM, VOCAB), jnp.float32),
        pltpu.VMEM((2, COLS), jnp.int32),
        pltpu.VMEM((2, EMBED_DIM, COLS), jnp.float32),
        pltpu.SemaphoreType.DMA,
        pltpu.SemaphoreType.DMA,
    ],
)
def _sc_embed(xt_hbm, tablet_hbm, out_hbm, tablet_v, idx_v, stage, w0, w1):
    wid = lax.axis_index("s") * NUM_CORES + lax.axis_index("c")
    col0 = wid * COLS

    pltpu.sync_copy(tablet_hbm, tablet_v)

    def gather_h(h, bsel):
        pltpu.sync_copy(xt_hbm.at[h, pl.ds(col0, COLS)], idx_v.at[bsel])

        @plsc.parallel_loop(0, NGROUP, unroll=2)
        def group_body(g):
            iv = idx_v[bsel, pl.ds(g * 16, 16)]
            for d in range(EMBED_DIM):
                vals = plsc.load_gather(tablet_v, [jnp.full((16,), d, jnp.int32), iv])
                stage[bsel, d, pl.ds(g * 16, 16)] = vals

    def write_h(h, bsel, sem):
        pltpu.async_copy(
            stage.at[bsel],
            out_hbm.at[pl.ds(h * EMBED_DIM, EMBED_DIM), pl.ds(col0, COLS)],
            sem,
        )

    def drain(sem):
        pltpu.make_async_copy(
            stage.at[0],
            out_hbm.at[pl.ds(0, EMBED_DIM), pl.ds(col0, COLS)],
            sem,
        ).wait()

    def pair_body(p, carry):
        h0 = p * 2

        @pl.when(p > 0)
        def _():
            drain(w0)

        gather_h(h0, 0)
        write_h(h0, 0, w0)

        @pl.when(p > 0)
        def _():
            drain(w1)

        gather_h(h0 + 1, 1)
        write_h(h0 + 1, 1, w1)
        return carry

    lax.fori_loop(0, NPAIR, pair_body, 0)
    drain(w0)
    drain(w1)


def kernel(x, item_id_table):
    out = _sc_embed(x.T, item_id_table.T)
    return out.T.reshape(BATCH, HIST, EMBED_DIM)


# speedup vs baseline: 28.9173x; 1.0006x over previous
import functools

import jax
import jax.numpy as jnp
from jax import lax
from jax.experimental import pallas as pl
from jax.experimental.pallas import tpu as pltpu
from jax.experimental.pallas import tpu_sc as plsc

VOCAB = 1000
EMBED_DIM = 32
BATCH = 16384
HIST = 50
OROWS = HIST * EMBED_DIM

NUM_CORES = 2
NUM_SUBCORES = 16
NW = NUM_CORES * NUM_SUBCORES
COLS = BATCH // NW
NGROUP = COLS // 16
NPAIR = HIST // 2

_mesh = plsc.VectorSubcoreMesh(core_axis_name="c", subcore_axis_name="s")


@functools.partial(
    pl.kernel,
    mesh=_mesh,
    out_type=jax.ShapeDtypeStruct((OROWS, BATCH), jnp.float32),
    compiler_params=pltpu.CompilerParams(needs_layout_passes=False),
    scratch_types=[
        pltpu.VMEM((EMBED_DIM * VOCAB,), jnp.float32),
        pltpu.VMEM((2, COLS), jnp.int32),
        pltpu.VMEM((2, EMBED_DIM, COLS), jnp.float32),
        pltpu.SemaphoreType.DMA,
        pltpu.SemaphoreType.DMA,
        pltpu.SemaphoreType.DMA,
    ],
)
def _sc_embed(xt_hbm, tablet_hbm, out_hbm, tablet_v, idx_v, stage, w0, w1, tsem):
    wid = lax.axis_index("s") * NUM_CORES + lax.axis_index("c")
    col0 = wid * COLS

    pltpu.async_copy(tablet_hbm, tablet_v, tsem).wait()

    def gather_h(h, bsel):
        pltpu.sync_copy(xt_hbm.at[h, pl.ds(col0, COLS)], idx_v.at[bsel])

        @plsc.parallel_loop(0, NGROUP, unroll=2)
        def group_body(g):
            iv = idx_v[bsel, pl.ds(g * 16, 16)]
            for d in range(EMBED_DIM):
                vals = plsc.load_gather(tablet_v, [iv + d * VOCAB])
                stage[bsel, d, pl.ds(g * 16, 16)] = vals

    def write_h(h, bsel, sem):
        pltpu.async_copy(
            stage.at[bsel],
            out_hbm.at[pl.ds(h * EMBED_DIM, EMBED_DIM), pl.ds(col0, COLS)],
            sem,
        )

    def drain(sem):
        pltpu.make_async_copy(
            stage.at[0],
            out_hbm.at[pl.ds(0, EMBED_DIM), pl.ds(col0, COLS)],
            sem,
        ).wait()

    def pair_body(p, carry):
        h0 = p * 2

        @pl.when(p > 0)
        def _():
            drain(w0)

        gather_h(h0, 0)
        write_h(h0, 0, w0)

        @pl.when(p > 0)
        def _():
            drain(w1)

        gather_h(h0 + 1, 1)
        write_h(h0 + 1, 1, w1)
        return carry

    lax.fori_loop(0, NPAIR, pair_body, 0)
    drain(w0)
    drain(w1)


def kernel(x, item_id_table):
    out = _sc_embed(x.T, item_id_table.T.reshape(EMBED_DIM * VOCAB))
    return out.T.reshape(BATCH, HIST, EMBED_DIM)


# speedup vs baseline: 29.6414x; 1.0250x over previous
import functools

import jax
import jax.numpy as jnp
from jax import lax
from jax.experimental import pallas as pl
from jax.experimental.pallas import tpu as pltpu
from jax.experimental.pallas import tpu_sc as plsc

VOCAB = 1000
EMBED_DIM = 32
BATCH = 16384
HIST = 50
OROWS = HIST * EMBED_DIM

NUM_CORES = 2
NUM_SUBCORES = 16
NW = NUM_CORES * NUM_SUBCORES
COLS = BATCH // NW
NGROUP = COLS // 16
NPAIR = HIST // 2

_mesh = plsc.VectorSubcoreMesh(core_axis_name="c", subcore_axis_name="s")


@functools.partial(
    pl.kernel,
    mesh=_mesh,
    out_type=jax.ShapeDtypeStruct((OROWS, BATCH), jnp.float32),
    compiler_params=pltpu.CompilerParams(needs_layout_passes=False),
    scratch_types=[
        pltpu.VMEM((EMBED_DIM * VOCAB,), jnp.float32),
        pltpu.VMEM((HIST, COLS), jnp.int32),
        pltpu.VMEM((2, EMBED_DIM, COLS), jnp.float32),
        pltpu.SemaphoreType.DMA,
        pltpu.SemaphoreType.DMA,
        pltpu.SemaphoreType.DMA,
    ],
)
def _sc_embed(xt_hbm, tablet_hbm, out_hbm, tablet_v, idx_v, stage, w0, w1, tsem):
    wid = lax.axis_index("s") * NUM_CORES + lax.axis_index("c")
    col0 = wid * COLS

    th = pltpu.async_copy(tablet_hbm, tablet_v, tsem)
    ih = pltpu.async_copy(xt_hbm.at[:, pl.ds(col0, COLS)], idx_v, tsem)
    th.wait()
    ih.wait()

    def gather_h(h, bsel):
        @plsc.parallel_loop(0, NGROUP, unroll=4)
        def group_body(g):
            iv = idx_v[h, pl.ds(g * 16, 16)]
            for d in range(EMBED_DIM):
                vals = plsc.load_gather(tablet_v, [iv + d * VOCAB])
                stage[bsel, d, pl.ds(g * 16, 16)] = vals

    def write_h(h, bsel, sem):
        pltpu.async_copy(
            stage.at[bsel],
            out_hbm.at[pl.ds(h * EMBED_DIM, EMBED_DIM), pl.ds(col0, COLS)],
            sem,
        )

    def drain(sem):
        pltpu.make_async_copy(
            stage.at[0],
            out_hbm.at[pl.ds(0, EMBED_DIM), pl.ds(col0, COLS)],
            sem,
        ).wait()

    def pair_body(p, carry):
        h0 = p * 2

        @pl.when(p > 0)
        def _():
            drain(w0)

        gather_h(h0, 0)
        write_h(h0, 0, w0)

        @pl.when(p > 0)
        def _():
            drain(w1)

        gather_h(h0 + 1, 1)
        write_h(h0 + 1, 1, w1)
        return carry

    lax.fori_loop(0, NPAIR, pair_body, 0)
    drain(w0)
    drain(w1)


def kernel(x, item_id_table):
    out = _sc_embed(x.T, item_id_table.T.reshape(EMBED_DIM * VOCAB))
    return out.T.reshape(BATCH, HIST, EMBED_DIM)
